# Initial kernel scaffold; baseline (speedup 1.0000x reference)
import jax
import jax.numpy as jnp
from jax.experimental import pallas as pl


def kernel(xyz, features, fc1_W, fc1_b, fc2_W, fc2_b, fcd1_W, fcd1_b, fcd2_W, fcd2_b, fcg1_W, fcg1_b, fcg2_W, fcg2_b, wq_W, wk_W, wv_W, fc1c_W, fc1c_b, fc2c_W, fc2c_b, wqc_W, wkc_W, wvc_W):
    raise NotImplementedError("write your pallas kernel here")


# speedup vs baseline: 3.8879x; 3.8879x over previous
import functools

import jax
import jax.numpy as jnp
from jax import lax
from jax.experimental import pallas as pl
from jax.experimental.pallas import tpu as pltpu

K = 16
P_BLK = 64

_HI = lax.Precision.HIGHEST


def _mm(a, b, precision=_HI):
    return lax.dot_general(a, b, (((1,), (0,)), ((), ())),
                           precision=precision,
                           preferred_element_type=jnp.float32)


def _mm_t(a, b, precision=_HI):
    return lax.dot_general(a, b, (((1,), (1,)), ((), ())),
                           precision=precision,
                           preferred_element_type=jnp.float32)


def _mmbf(a, b):
    return lax.dot_general(a.astype(jnp.bfloat16), b.astype(jnp.bfloat16),
                           (((1,), (0,)), ((), ())),
                           preferred_element_type=jnp.float32)


def _mmbf_t(a, b):
    return lax.dot_general(a.astype(jnp.bfloat16), b.astype(jnp.bfloat16),
                           (((1,), (1,)), ((), ())),
                           preferred_element_type=jnp.float32)


def _stage_a(pts_ref, ptst_ref, feat_ref, clsf_ref,
             fc1_t, fc1_b, wq_t, wk_t, wv_t,
             fc1c_t, fc1c_b, wqc_t, wkc_t, wvc_t, fc2c_t, fc2c_b,
             idx_ref, q_ref, k_ref, v_ref, clsout_ref, clsattn_ref):
    pts = pts_ref[0]
    ptst = ptst_ref[0]
    n = pts.shape[0]
    g = _mmbf_t(pts, pts)
    pts2 = pts * pts
    n_row = jnp.sum(pts2, axis=1, keepdims=True)
    n_col = jnp.sum(ptst * ptst, axis=0, keepdims=True)
    d = n_row + n_col - 2.0 * g

    lane = lax.broadcasted_iota(jnp.int32, (n, n), 1)
    cols = []
    for _ in range(K):
        m = jnp.min(d, axis=1, keepdims=True)
        am = jnp.min(jnp.where(d == m, lane, n), axis=1, keepdims=True)
        cols.append(am)
        d = jnp.where(lane == am, jnp.inf, d)
    idx_ref[0] = jnp.concatenate(cols, axis=1)

    x = _mmbf(feat_ref[0], fc1_t[...]) + fc1_b[...]
    q_ref[0] = _mmbf(x, wq_t[...])
    k_ref[0] = _mmbf(x, wk_t[...])
    v_ref[0] = _mmbf(x, wv_t[...])

    kc = _mmbf(x, wkc_t[...])
    vc = _mmbf(x, wvc_t[...])
    cf = clsf_ref[0]
    cx = _mmbf(cf, fc1c_t[...]) + fc1c_b[...]
    qc = _mmbf(cx, wqc_t[...])
    dm = qc.shape[1]
    logits = _mmbf_t(qc, kc) / jnp.sqrt(jnp.float32(dm))
    lm = jnp.max(logits, axis=1, keepdims=True)
    e = jnp.exp(logits - lm)
    ca = e / jnp.sum(e, axis=1, keepdims=True)
    clsattn_ref[0] = ca
    cr = _mmbf(ca, vc)
    clsout_ref[0] = _mmbf(cr, fc2c_t[...]) + fc2c_b[...] + cf


def _stage_b(idx_ref, q_ref, feat_ref, k_ref, v_ref, pts_ref,
             fcd1_t, fcd1_b, fcd2_t, fcd2_b,
             fcg1_t, fcg1_b, fcg2_t, fcg2_b, fc2_t, fc2_b,
             attn_ref, res_ref):
    nblk = pl.program_id(1)
    kfull = k_ref[0]
    vfull = v_ref[0]
    pts = pts_ref[0]
    n = kfull.shape[0]
    dm = kfull.shape[1]
    p = q_ref.shape[1]
    rows = p * K

    idxb = idx_ref[0]
    lane = lax.broadcasted_iota(jnp.int32, (rows, n), 1)
    oh_nbr = (idxb == lane).astype(jnp.float32)
    self_i = (lax.broadcasted_iota(jnp.int32, (rows, n), 0) // K) + nblk * p
    oh_self = (self_i == lane).astype(jnp.float32)

    delta = _mm(oh_self - oh_nbr, pts)
    kk = _mm(oh_nbr, kfull)
    vv = _mm(oh_nbr, vfull)

    h = jnp.maximum(_mm(delta, fcd1_t[...]) + fcd1_b[...], 0.0)
    pos = _mmbf(h, fcd2_t[...]) + fcd2_b[...]

    gin = (pos - kk).reshape(p, K, dm) + q_ref[0][:, None, :]
    g = jnp.maximum(_mmbf(gin.reshape(rows, dm), fcg1_t[...]) + fcg1_b[...], 0.0)
    a = (_mmbf(g, fcg2_t[...]) + fcg2_b[...]) / jnp.sqrt(jnp.float32(dm))

    a3 = a.reshape(p, K, dm)
    am = jnp.max(a3, axis=1, keepdims=True)
    e = jnp.exp(a3 - am)
    attn = e / jnp.sum(e, axis=1, keepdims=True)
    attn_ref[0] = attn

    w = attn * (vv + pos).reshape(p, K, dm)
    resb = jnp.sum(w, axis=1)
    res_ref[0] = _mmbf(resb, fc2_t[...]) + fc2_b[...] + feat_ref[0]


def kernel(xyz, features, fc1_W, fc1_b, fc2_W, fc2_b, fcd1_W, fcd1_b,
           fcd2_W, fcd2_b, fcg1_W, fcg1_b, fcg2_W, fcg2_b, wq_W, wk_W, wv_W,
           fc1c_W, fc1c_b, fc2c_W, fc2c_b, wqc_W, wkc_W, wvc_W):
    b = xyz.shape[0]
    n = xyz.shape[1] - 1
    dp = features.shape[2]
    dm = fc1_W.shape[0]

    pts = xyz[:, 1:, :]
    clsf = features[:, :1, :]
    feat = features[:, 1:, :]

    r2 = lambda w: w.reshape(1, -1)

    spec_w2 = lambda s: pl.BlockSpec(s, lambda i: (0, 0))
    grid_a = (b,)
    a_in_specs = [
        pl.BlockSpec((1, n, 3), lambda i: (i, 0, 0)),
        pl.BlockSpec((1, 3, n), lambda i: (i, 0, 0)),
        pl.BlockSpec((1, n, dp), lambda i: (i, 0, 0)),
        pl.BlockSpec((1, 1, dp), lambda i: (i, 0, 0)),
        spec_w2((dp, dm)), spec_w2((1, dm)),
        spec_w2((dm, dm)), spec_w2((dm, dm)), spec_w2((dm, dm)),
        spec_w2((dp, dm)), spec_w2((1, dm)),
        spec_w2((dm, dm)), spec_w2((dm, dm)), spec_w2((dm, dm)),
        spec_w2((dm, dp)), spec_w2((1, dp)),
    ]
    a_out_specs = [
        pl.BlockSpec((1, n, K), lambda i: (i, 0, 0)),
        pl.BlockSpec((1, n, dm), lambda i: (i, 0, 0)),
        pl.BlockSpec((1, n, dm), lambda i: (i, 0, 0)),
        pl.BlockSpec((1, n, dm), lambda i: (i, 0, 0)),
        pl.BlockSpec((1, 1, dp), lambda i: (i, 0, 0)),
        pl.BlockSpec((1, 1, n), lambda i: (i, 0, 0)),
    ]
    a_out_shapes = [
        jax.ShapeDtypeStruct((b, n, K), jnp.int32),
        jax.ShapeDtypeStruct((b, n, dm), jnp.float32),
        jax.ShapeDtypeStruct((b, n, dm), jnp.float32),
        jax.ShapeDtypeStruct((b, n, dm), jnp.float32),
        jax.ShapeDtypeStruct((b, 1, dp), jnp.float32),
        jax.ShapeDtypeStruct((b, 1, n), jnp.float32),
    ]
    idx, q, kmat, vmat, cls_out, cls_attn = pl.pallas_call(
        _stage_a,
        grid=grid_a,
        in_specs=a_in_specs,
        out_specs=a_out_specs,
        out_shape=a_out_shapes,
        compiler_params=pltpu.CompilerParams(
            dimension_semantics=("parallel",)),
    )(pts, jnp.swapaxes(pts, 1, 2), feat, clsf,
      fc1_W.T, r2(fc1_b), wq_W.T, wk_W.T, wv_W.T,
      fc1c_W.T, r2(fc1c_b), wqc_W.T, wkc_W.T, wvc_W.T,
      fc2c_W.T, r2(fc2c_b))

    idx_flat = idx.reshape(b, n * K, 1)

    p = P_BLK
    nblk = n // p
    rows = p * K
    b_in_specs = [
        pl.BlockSpec((1, rows, 1), lambda i, j: (i, j, 0)),
        pl.BlockSpec((1, p, dm), lambda i, j: (i, j, 0)),
        pl.BlockSpec((1, p, dp), lambda i, j: (i, j, 0)),
        pl.BlockSpec((1, n, dm), lambda i, j: (i, 0, 0)),
        pl.BlockSpec((1, n, dm), lambda i, j: (i, 0, 0)),
        pl.BlockSpec((1, n, 3), lambda i, j: (i, 0, 0)),
        pl.BlockSpec((3, dm), lambda i, j: (0, 0)),
        pl.BlockSpec((1, dm), lambda i, j: (0, 0)),
        pl.BlockSpec((dm, dm), lambda i, j: (0, 0)),
        pl.BlockSpec((1, dm), lambda i, j: (0, 0)),
        pl.BlockSpec((dm, dm), lambda i, j: (0, 0)),
        pl.BlockSpec((1, dm), lambda i, j: (0, 0)),
        pl.BlockSpec((dm, dm), lambda i, j: (0, 0)),
        pl.BlockSpec((1, dm), lambda i, j: (0, 0)),
        pl.BlockSpec((dm, dp), lambda i, j: (0, 0)),
        pl.BlockSpec((1, dp), lambda i, j: (0, 0)),
    ]
    b_out_specs = [
        pl.BlockSpec((1, p, K, dm), lambda i, j: (i, j, 0, 0)),
        pl.BlockSpec((1, p, dp), lambda i, j: (i, j, 0)),
    ]
    b_out_shapes = [
        jax.ShapeDtypeStruct((b, n, K, dm), jnp.float32),
        jax.ShapeDtypeStruct((b, n, dp), jnp.float32),
    ]
    attn, res = pl.pallas_call(
        _stage_b,
        grid=(b, nblk),
        in_specs=b_in_specs,
        out_specs=b_out_specs,
        out_shape=b_out_shapes,
        compiler_params=pltpu.CompilerParams(
            dimension_semantics=("parallel", "parallel")),
    )(idx_flat, q, feat, kmat, vmat, pts,
      fcd1_W.T, r2(fcd1_b), fcd2_W.T, r2(fcd2_b),
      fcg1_W.T, r2(fcg1_b), fcg2_W.T, r2(fcg2_b),
      fc2_W.T, r2(fc2_b))

    out = jnp.concatenate([cls_out, res], axis=1)
    return out, attn, cls_attn


# speedup vs baseline: 7.2225x; 1.8577x over previous
import jax
import jax.numpy as jnp
from jax import lax
from jax.experimental import pallas as pl
from jax.experimental.pallas import tpu as pltpu

K = 16
P_BLK = 128

_HI = lax.Precision.HIGHEST


def _mm(a, b, precision=_HI):
    return lax.dot_general(a, b, (((1,), (0,)), ((), ())),
                           precision=precision,
                           preferred_element_type=jnp.float32)


def _mmbf(a, b):
    return lax.dot_general(a.astype(jnp.bfloat16), b.astype(jnp.bfloat16),
                           (((1,), (0,)), ((), ())),
                           preferred_element_type=jnp.float32)


def _mmbf_t(a, b):
    return lax.dot_general(a.astype(jnp.bfloat16), b.astype(jnp.bfloat16),
                           (((1,), (1,)), ((), ())),
                           preferred_element_type=jnp.float32)


def _split(x):
    hi = x.astype(jnp.bfloat16)
    lo = (x - hi.astype(jnp.float32)).astype(jnp.bfloat16)
    return hi, lo


def _stage_a(pts_ref, ptst_ref, feat_ref, clsf_ref,
             fc1_t, wq_t, wk_t, wv_t,
             fc1c_t, wqc_t, wkc_t, wvc_t, fc2c_t,
             idx_ref, q_ref, khi_ref, vhi_ref,
             clsout_ref, clsattn_ref):
    pts = pts_ref[0]
    ptst = ptst_ref[0]
    n = pts.shape[0]
    g = _mmbf_t(pts, pts)
    pts2 = pts * pts
    n_row = jnp.sum(pts2, axis=1, keepdims=True)
    n_col = jnp.sum(ptst * ptst, axis=0, keepdims=True)
    d = n_row + n_col - 2.0 * g

    lane = lax.broadcasted_iota(jnp.int32, (n, n), 1)
    cols = []
    for _ in range(K):
        am = jnp.argmin(d, axis=1).astype(jnp.int32).reshape(n, 1)
        cols.append(am)
        d = jnp.where(lane == am, jnp.inf, d)
    idx_ref[0] = jnp.concatenate(cols, axis=1)

    x = _mmbf(feat_ref[0], fc1_t[...])
    q_ref[0] = _mmbf(x, wq_t[...])
    khi_ref[0] = _mmbf(x, wk_t[...]).astype(jnp.bfloat16)
    vhi_ref[0] = _mmbf(x, wv_t[...]).astype(jnp.bfloat16)

    kc = _mmbf(x, wkc_t[...])
    vc = _mmbf(x, wvc_t[...])
    cf = clsf_ref[0]
    cx = _mmbf(cf, fc1c_t[...])
    qc = _mmbf(cx, wqc_t[...])
    dm = qc.shape[1]
    logits = _mmbf_t(qc, kc) / jnp.sqrt(jnp.float32(dm))
    lm = jnp.max(logits, axis=1, keepdims=True)
    e = jnp.exp(logits - lm)
    ca = e / jnp.sum(e, axis=1, keepdims=True)
    clsattn_ref[0] = ca
    cr = _mmbf(ca, vc)
    clsout_ref[0] = _mmbf(cr, fc2c_t[...]) + cf


def _stage_b(idx_ref, q_ref, feat_ref, ptsb_ref,
             khi_ref, vhi_ref, phi_ref, plo_ref,
             fcd1_hi, fcd1_lo, fcd2_t,
             fcg1_t, fcg2_t, fc2_t,
             attn_ref, res_ref):
    n = khi_ref.shape[1]
    dm = khi_ref.shape[2]
    p = q_ref.shape[1]
    rows = p * K

    idxb = idx_ref[0].astype(jnp.int16)
    lane = lax.broadcasted_iota(jnp.int16, (rows, n), 1)
    oh = jnp.where(idxb == lane, jnp.bfloat16(1.0), jnp.bfloat16(0.0))

    kk = _mmbf(oh, khi_ref[0])
    vv = _mmbf(oh, vhi_ref[0])
    pg = _mmbf(oh, phi_ref[0]) + _mmbf(oh, plo_ref[0])

    ptsb = ptsb_ref[0]
    delta = jnp.broadcast_to(ptsb[:, None, :], (p, K, 3)).reshape(rows, 3) - pg
    dhi, dlo = _split(delta)
    h = jnp.maximum(_mmbf(dhi, fcd1_hi[...])
                    + (_mmbf(dhi, fcd1_lo[...]) + _mmbf(dlo, fcd1_hi[...])),
                    0.0)
    pos = _mmbf(h, fcd2_t[...])

    gin = (pos - kk).reshape(p, K, dm) + q_ref[0][:, None, :]
    g = jnp.maximum(_mmbf(gin.reshape(rows, dm), fcg1_t[...]), 0.0)
    a = _mmbf(g, fcg2_t[...]) / jnp.sqrt(jnp.float32(dm))

    a3 = a.reshape(p, K, dm)
    am = jnp.max(a3, axis=1, keepdims=True)
    e = jnp.exp(a3 - am)
    attn = e * (1.0 / jnp.sum(e, axis=1, keepdims=True))
    attn_ref[0] = attn

    w = attn * (vv + pos).reshape(p, K, dm)
    resb = jnp.sum(w, axis=1)
    res_ref[0] = _mmbf(resb, fc2_t[...]) + feat_ref[0]


def kernel(xyz, features, fc1_W, fc1_b, fc2_W, fc2_b, fcd1_W, fcd1_b,
           fcd2_W, fcd2_b, fcg1_W, fcg1_b, fcg2_W, fcg2_b, wq_W, wk_W, wv_W,
           fc1c_W, fc1c_b, fc2c_W, fc2c_b, wqc_W, wkc_W, wvc_W):
    b = xyz.shape[0]
    n = xyz.shape[1] - 1
    dp = features.shape[2]
    dm = fc1_W.shape[0]

    pts = xyz[:, 1:, :]
    clsf = features[:, :1, :]
    feat = features[:, 1:, :]

    bf = lambda w: w.T.astype(jnp.bfloat16)
    r2 = lambda w: w.reshape(1, -1)
    phi = pts.astype(jnp.bfloat16)
    plo = (pts - phi.astype(jnp.float32)).astype(jnp.bfloat16)
    fcd1_hi = fcd1_W.T.astype(jnp.bfloat16)
    fcd1_lo = (fcd1_W.T - fcd1_hi.astype(jnp.float32)).astype(jnp.bfloat16)

    spec_w2 = lambda s: pl.BlockSpec(s, lambda i: (0, 0))
    a_in_specs = [
        pl.BlockSpec((1, n, 3), lambda i: (i, 0, 0)),
        pl.BlockSpec((1, 3, n), lambda i: (i, 0, 0)),
        pl.BlockSpec((1, n, dp), lambda i: (i, 0, 0)),
        pl.BlockSpec((1, 1, dp), lambda i: (i, 0, 0)),
        spec_w2((dp, dm)),
        spec_w2((dm, dm)), spec_w2((dm, dm)), spec_w2((dm, dm)),
        spec_w2((dp, dm)),
        spec_w2((dm, dm)), spec_w2((dm, dm)), spec_w2((dm, dm)),
        spec_w2((dm, dp)),
    ]
    a_out_specs = [
        pl.BlockSpec((1, n, K), lambda i: (i, 0, 0)),
        pl.BlockSpec((1, n, dm), lambda i: (i, 0, 0)),
        pl.BlockSpec((1, n, dm), lambda i: (i, 0, 0)),
        pl.BlockSpec((1, n, dm), lambda i: (i, 0, 0)),
        pl.BlockSpec((1, 1, dp), lambda i: (i, 0, 0)),
        pl.BlockSpec((1, 1, n), lambda i: (i, 0, 0)),
    ]
    a_out_shapes = [
        jax.ShapeDtypeStruct((b, n, K), jnp.int32),
        jax.ShapeDtypeStruct((b, n, dm), jnp.float32),
        jax.ShapeDtypeStruct((b, n, dm), jnp.bfloat16),
        jax.ShapeDtypeStruct((b, n, dm), jnp.bfloat16),
        jax.ShapeDtypeStruct((b, 1, dp), jnp.float32),
        jax.ShapeDtypeStruct((b, 1, n), jnp.float32),
    ]
    (idx, q, khi, vhi, cls_out, cls_attn) = pl.pallas_call(
        _stage_a,
        grid=(b,),
        in_specs=a_in_specs,
        out_specs=a_out_specs,
        out_shape=a_out_shapes,
        compiler_params=pltpu.CompilerParams(
            dimension_semantics=("parallel",)),
    )(pts, jnp.swapaxes(pts, 1, 2), feat, clsf,
      bf(fc1_W), bf(wq_W), bf(wk_W), bf(wv_W),
      bf(fc1c_W), bf(wqc_W), bf(wkc_W), bf(wvc_W),
      bf(fc2c_W))

    idx_flat = idx.reshape(b, n * K, 1)

    p = P_BLK
    nblk = n // p
    rows = p * K
    spec_bw = lambda s: pl.BlockSpec(s, lambda i, j: (0, 0))
    spec_tbl = lambda last: pl.BlockSpec((1, n, last), lambda i, j: (i, 0, 0))
    b_in_specs = [
        pl.BlockSpec((1, rows, 1), lambda i, j: (i, j, 0)),
        pl.BlockSpec((1, p, dm), lambda i, j: (i, j, 0)),
        pl.BlockSpec((1, p, dp), lambda i, j: (i, j, 0)),
        pl.BlockSpec((1, p, 3), lambda i, j: (i, j, 0)),
        spec_tbl(dm), spec_tbl(dm),
        spec_tbl(3), spec_tbl(3),
        spec_bw((3, dm)), spec_bw((3, dm)),
        spec_bw((dm, dm)),
        spec_bw((dm, dm)),
        spec_bw((dm, dm)),
        spec_bw((dm, dp)),
    ]
    b_out_specs = [
        pl.BlockSpec((1, p, K, dm), lambda i, j: (i, j, 0, 0)),
        pl.BlockSpec((1, p, dp), lambda i, j: (i, j, 0)),
    ]
    b_out_shapes = [
        jax.ShapeDtypeStruct((b, n, K, dm), jnp.float32),
        jax.ShapeDtypeStruct((b, n, dp), jnp.float32),
    ]
    attn, res = pl.pallas_call(
        _stage_b,
        grid=(b, nblk),
        in_specs=b_in_specs,
        out_specs=b_out_specs,
        out_shape=b_out_shapes,
        compiler_params=pltpu.CompilerParams(
            dimension_semantics=("parallel", "parallel")),
    )(idx_flat, q, feat, pts,
      khi, vhi, phi, plo,
      fcd1_hi, fcd1_lo, bf(fcd2_W),
      bf(fcg1_W), bf(fcg2_W),
      bf(fc2_W))

    out = jnp.concatenate([cls_out, res], axis=1)
    return out, attn, cls_attn


# speedup vs baseline: 8.3933x; 1.1621x over previous
import jax
import jax.numpy as jnp
from jax import lax
from jax.experimental import pallas as pl
from jax.experimental.pallas import tpu as pltpu

K = 16
P_BLK = 128

_HI = lax.Precision.HIGHEST


def _mm(a, b, precision=_HI):
    return lax.dot_general(a, b, (((1,), (0,)), ((), ())),
                           precision=precision,
                           preferred_element_type=jnp.float32)


def _mmbf(a, b):
    return lax.dot_general(a.astype(jnp.bfloat16), b.astype(jnp.bfloat16),
                           (((1,), (0,)), ((), ())),
                           preferred_element_type=jnp.float32)


def _mmbf_t(a, b):
    return lax.dot_general(a.astype(jnp.bfloat16), b.astype(jnp.bfloat16),
                           (((1,), (1,)), ((), ())),
                           preferred_element_type=jnp.float32)


def _split(x):
    hi = x.astype(jnp.bfloat16)
    lo = (x - hi.astype(jnp.float32)).astype(jnp.bfloat16)
    return hi, lo


def _stage_a(pts_ref, ptst_ref, feat_ref, clsf_ref,
             fc1_t, wq_t, wk_t, wv_t,
             fc1c_t, wqc_t, wkc_t, wvc_t, fc2c_t,
             idx_ref, q_ref, khi_ref, vhi_ref,
             clsout_ref, clsattn_ref):
    pts = pts_ref[0]
    ptst = ptst_ref[0]
    n = pts.shape[0]
    g = _mmbf_t(pts, pts)
    pts2 = pts * pts
    n_row = jnp.sum(pts2, axis=1, keepdims=True)
    n_col = jnp.sum(ptst * ptst, axis=0, keepdims=True)
    d = n_row + n_col - 2.0 * g

    lane = lax.broadcasted_iota(jnp.int32, (n, n), 1)
    cols = []
    for _ in range(K):
        am = jnp.argmin(d, axis=1).astype(jnp.int32).reshape(n, 1)
        cols.append(am)
        d = jnp.where(lane == am, jnp.inf, d)
    idx_ref[0] = jnp.concatenate(cols, axis=1)

    x = _mmbf(feat_ref[0], fc1_t[...])
    q_ref[0] = _mmbf(x, wq_t[...])
    khi_ref[0] = _mmbf(x, wk_t[...]).astype(jnp.bfloat16)
    vhi_ref[0] = _mmbf(x, wv_t[...]).astype(jnp.bfloat16)

    kc = _mmbf(x, wkc_t[...])
    vc = _mmbf(x, wvc_t[...])
    cf = clsf_ref[0]
    cx = _mmbf(cf, fc1c_t[...])
    qc = _mmbf(cx, wqc_t[...])
    dm = qc.shape[1]
    logits = _mmbf_t(qc, kc) / jnp.sqrt(jnp.float32(dm))
    lm = jnp.max(logits, axis=1, keepdims=True)
    e = jnp.exp(logits - lm)
    ca = e / jnp.sum(e, axis=1, keepdims=True)
    clsattn_ref[0] = ca
    cr = _mmbf(ca, vc)
    clsout_ref[0] = _mmbf(cr, fc2c_t[...]) + cf


def _stage_b(idx_ref, q_ref, feat_ref, ptsb_ref,
             khi_ref, vhi_ref, phi_ref, plo_ref,
             fcd1_cat, fcd2_t,
             fcg1_t, fcg2_t, fc2_t,
             attn_ref, res_ref):
    n = khi_ref.shape[1]
    dm = khi_ref.shape[2]
    p = q_ref.shape[1]
    rows = p * K

    idxb = idx_ref[0].astype(jnp.int16)
    lane = lax.broadcasted_iota(jnp.int16, (rows, n), 1)
    oh = jnp.where(idxb == lane, jnp.bfloat16(1.0), jnp.bfloat16(0.0))

    kk = _mmbf(oh, khi_ref[0])
    vv = _mmbf(oh, vhi_ref[0])
    pg = _mmbf(oh, phi_ref[0]) + _mmbf(oh, plo_ref[0])

    ptsb = ptsb_ref[0]
    delta = jnp.broadcast_to(ptsb[:, None, :], (p, K, 3)).reshape(rows, 3) - pg
    dhi, dlo = _split(delta)
    dcat = jnp.concatenate([dhi, dhi, dlo], axis=1)
    h = jnp.maximum(_mmbf(dcat, fcd1_cat[...]), 0.0)
    pos = _mmbf(h, fcd2_t[...])

    gin = (pos - kk).reshape(p, K, dm) + q_ref[0][:, None, :]
    g = jnp.maximum(_mmbf(gin.reshape(rows, dm), fcg1_t[...]), 0.0)
    a = _mmbf(g, fcg2_t[...]) / jnp.sqrt(jnp.float32(dm))

    a3 = a.reshape(p, K, dm)
    am = jnp.max(a3, axis=1, keepdims=True)
    e = jnp.exp(a3 - am)
    attn = e * (1.0 / jnp.sum(e, axis=1, keepdims=True))
    attn_ref[0] = attn

    w = attn * (vv + pos).reshape(p, K, dm)
    resb = jnp.sum(w, axis=1)
    res_ref[0] = _mmbf(resb, fc2_t[...]) + feat_ref[0]


def kernel(xyz, features, fc1_W, fc1_b, fc2_W, fc2_b, fcd1_W, fcd1_b,
           fcd2_W, fcd2_b, fcg1_W, fcg1_b, fcg2_W, fcg2_b, wq_W, wk_W, wv_W,
           fc1c_W, fc1c_b, fc2c_W, fc2c_b, wqc_W, wkc_W, wvc_W):
    b = xyz.shape[0]
    n = xyz.shape[1] - 1
    dp = features.shape[2]
    dm = fc1_W.shape[0]

    pts = xyz[:, 1:, :]
    clsf = features[:, :1, :]
    feat = features[:, 1:, :]

    bf = lambda w: w.T.astype(jnp.bfloat16)
    r2 = lambda w: w.reshape(1, -1)
    phi = pts.astype(jnp.bfloat16)
    plo = (pts - phi.astype(jnp.float32)).astype(jnp.bfloat16)
    fcd1_hi = fcd1_W.T.astype(jnp.bfloat16)
    fcd1_lo = (fcd1_W.T - fcd1_hi.astype(jnp.float32)).astype(jnp.bfloat16)
    fcd1_cat = jnp.concatenate([fcd1_hi, fcd1_lo, fcd1_hi], axis=0)

    spec_w2 = lambda s: pl.BlockSpec(s, lambda i: (0, 0))
    a_in_specs = [
        pl.BlockSpec((1, n, 3), lambda i: (i, 0, 0)),
        pl.BlockSpec((1, 3, n), lambda i: (i, 0, 0)),
        pl.BlockSpec((1, n, dp), lambda i: (i, 0, 0)),
        pl.BlockSpec((1, 1, dp), lambda i: (i, 0, 0)),
        spec_w2((dp, dm)),
        spec_w2((dm, dm)), spec_w2((dm, dm)), spec_w2((dm, dm)),
        spec_w2((dp, dm)),
        spec_w2((dm, dm)), spec_w2((dm, dm)), spec_w2((dm, dm)),
        spec_w2((dm, dp)),
    ]
    a_out_specs = [
        pl.BlockSpec((1, n, K), lambda i: (i, 0, 0)),
        pl.BlockSpec((1, n, dm), lambda i: (i, 0, 0)),
        pl.BlockSpec((1, n, dm), lambda i: (i, 0, 0)),
        pl.BlockSpec((1, n, dm), lambda i: (i, 0, 0)),
        pl.BlockSpec((1, 1, dp), lambda i: (i, 0, 0)),
        pl.BlockSpec((1, 1, n), lambda i: (i, 0, 0)),
    ]
    a_out_shapes = [
        jax.ShapeDtypeStruct((b, n, K), jnp.int32),
        jax.ShapeDtypeStruct((b, n, dm), jnp.float32),
        jax.ShapeDtypeStruct((b, n, dm), jnp.bfloat16),
        jax.ShapeDtypeStruct((b, n, dm), jnp.bfloat16),
        jax.ShapeDtypeStruct((b, 1, dp), jnp.float32),
        jax.ShapeDtypeStruct((b, 1, n), jnp.float32),
    ]
    (idx, q, khi, vhi, cls_out, cls_attn) = pl.pallas_call(
        _stage_a,
        grid=(b,),
        in_specs=a_in_specs,
        out_specs=a_out_specs,
        out_shape=a_out_shapes,
        compiler_params=pltpu.CompilerParams(
            dimension_semantics=("parallel",)),
    )(pts, jnp.swapaxes(pts, 1, 2), feat, clsf,
      bf(fc1_W), bf(wq_W), bf(wk_W), bf(wv_W),
      bf(fc1c_W), bf(wqc_W), bf(wkc_W), bf(wvc_W),
      bf(fc2c_W))

    idx_flat = idx.reshape(b, n * K, 1)

    p = P_BLK
    nblk = n // p
    rows = p * K
    spec_bw = lambda s: pl.BlockSpec(s, lambda i, j: (0, 0))
    spec_tbl = lambda last: pl.BlockSpec((1, n, last), lambda i, j: (i, 0, 0))
    b_in_specs = [
        pl.BlockSpec((1, rows, 1), lambda i, j: (i, j, 0)),
        pl.BlockSpec((1, p, dm), lambda i, j: (i, j, 0)),
        pl.BlockSpec((1, p, dp), lambda i, j: (i, j, 0)),
        pl.BlockSpec((1, p, 3), lambda i, j: (i, j, 0)),
        spec_tbl(dm), spec_tbl(dm),
        spec_tbl(3), spec_tbl(3),
        spec_bw((9, dm)),
        spec_bw((dm, dm)),
        spec_bw((dm, dm)),
        spec_bw((dm, dm)),
        spec_bw((dm, dp)),
    ]
    b_out_specs = [
        pl.BlockSpec((1, p, K, dm), lambda i, j: (i, j, 0, 0)),
        pl.BlockSpec((1, p, dp), lambda i, j: (i, j, 0)),
    ]
    b_out_shapes = [
        jax.ShapeDtypeStruct((b, n, K, dm), jnp.float32),
        jax.ShapeDtypeStruct((b, n, dp), jnp.float32),
    ]
    attn, res = pl.pallas_call(
        _stage_b,
        grid=(b, nblk),
        in_specs=b_in_specs,
        out_specs=b_out_specs,
        out_shape=b_out_shapes,
        compiler_params=pltpu.CompilerParams(
            dimension_semantics=("parallel", "parallel")),
    )(idx_flat, q, feat, pts,
      khi, vhi, phi, plo,
      fcd1_cat, bf(fcd2_W),
      bf(fcg1_W), bf(fcg2_W),
      bf(fc2_W))

    out = jnp.concatenate([cls_out, res], axis=1)
    return out, attn, cls_attn
